# speedup vs baseline: 1.9835x; 1.9835x over previous
import functools

import jax
import jax.numpy as jnp
from jax import lax
from jax.experimental import pallas as pl
from jax.experimental.pallas import tpu as pltpu
from jax.experimental.pallas import tpu_sc as plsc

EMBED_DIM = 128
MAX_LEN = 500
N = 16384

NUM_CORES = 2
NUM_SUBCORES = 16
NUM_WORKERS = NUM_CORES * NUM_SUBCORES
B_PER_W = N // NUM_WORKERS
CHUNK = 128
NCHUNKS = B_PER_W // CHUNK
LANES = 16
VECS_PER_ROW = EMBED_DIM // LANES


def _sc_body(xs_hbm, ys_hbm, pe_hbm, out_hbm, xs_v, ys_v, bx, by, sem_x, sem_y):
    wid = lax.axis_index("s") * NUM_CORES + lax.axis_index("c")
    base = wid * B_PER_W
    pltpu.sync_copy(xs_hbm.at[pl.ds(base, B_PER_W)], xs_v)
    pltpu.sync_copy(ys_hbm.at[pl.ds(base, B_PER_W)], ys_v)

    for c in range(NCHUNKS):
        cx = pltpu.async_copy(
            pe_hbm.at[xs_v.at[pl.ds(c * CHUNK, CHUNK)]], bx, sem_x
        )
        cy = pltpu.async_copy(
            pe_hbm.at[ys_v.at[pl.ds(c * CHUNK, CHUNK)]], by, sem_y
        )
        cx.wait()
        cy.wait()

        def row_step(r, _):
            for k in range(VECS_PER_ROW):
                sl = pl.ds(k * LANES, LANES)
                bx[r, sl] = (bx[r, sl] + by[r, sl]) * 0.5
            return ()

        lax.fori_loop(0, CHUNK, row_step, (), unroll=2)

        pltpu.sync_copy(bx, out_hbm.at[pl.ds(base + c * CHUNK, CHUNK)])


@jax.jit
def _pe_lookup(xs, ys, pe):
    mesh = plsc.VectorSubcoreMesh(core_axis_name="c", subcore_axis_name="s")
    return pl.kernel(
        _sc_body,
        mesh=mesh,
        out_type=jax.ShapeDtypeStruct((N, EMBED_DIM), jnp.float32),
        scratch_types=[
            pltpu.VMEM((B_PER_W,), jnp.int32),
            pltpu.VMEM((B_PER_W,), jnp.int32),
            pltpu.VMEM((CHUNK, EMBED_DIM), jnp.float32),
            pltpu.VMEM((CHUNK, EMBED_DIM), jnp.float32),
            pltpu.SemaphoreType.DMA,
            pltpu.SemaphoreType.DMA,
        ],
    )(xs, ys, pe)


def kernel(coords, pe):
    xs = coords[:, 0].astype(jnp.int32)
    ys = coords[:, 1].astype(jnp.int32)
    return _pe_lookup(xs, ys, pe)


# speedup vs baseline: 2.1374x; 1.0775x over previous
import functools

import jax
import jax.numpy as jnp
from jax import lax
from jax.experimental import pallas as pl
from jax.experimental.pallas import tpu as pltpu
from jax.experimental.pallas import tpu_sc as plsc

EMBED_DIM = 128
MAX_LEN = 500
N = 16384

NUM_CORES = 2
NUM_SUBCORES = 16
NUM_WORKERS = NUM_CORES * NUM_SUBCORES
B_PER_W = N // NUM_WORKERS
CHUNK = 128
NCHUNKS = B_PER_W // CHUNK
LANES = 16
VECS_PER_ROW = EMBED_DIM // LANES


def _sc_body(
    xs_hbm, ys_hbm, pe_hbm, out_hbm, xs_v, ys_v,
    bx0, by0, bx1, by1, sx0, sy0, sx1, sy1, so0, so1,
):
    bx = (bx0, bx1)
    by = (by0, by1)
    sx = (sx0, sx1)
    sy = (sy0, sy1)
    so = (so0, so1)

    wid = lax.axis_index("s") * NUM_CORES + lax.axis_index("c")
    base = wid * B_PER_W
    pltpu.sync_copy(xs_hbm.at[pl.ds(base, B_PER_W)], xs_v)
    pltpu.sync_copy(ys_hbm.at[pl.ds(base, B_PER_W)], ys_v)

    def start_gathers(c):
        p = c % 2
        gx = pltpu.async_copy(
            pe_hbm.at[xs_v.at[pl.ds(c * CHUNK, CHUNK)]], bx[p], sx[p]
        )
        gy = pltpu.async_copy(
            pe_hbm.at[ys_v.at[pl.ds(c * CHUNK, CHUNK)]], by[p], sy[p]
        )
        return gx, gy

    pend_g = {0: start_gathers(0)}
    pend_out = [None, None]
    for c in range(NCHUNKS):
        p = c % 2
        if c + 1 < NCHUNKS:
            q = (c + 1) % 2
            if pend_out[q] is not None:
                pend_out[q].wait()
                pend_out[q] = None
            pend_g[c + 1] = start_gathers(c + 1)
        gx, gy = pend_g.pop(c)
        gx.wait()
        gy.wait()

        def row_step(r, _, p=p):
            for k in range(VECS_PER_ROW):
                sl = pl.ds(k * LANES, LANES)
                bx[p][r, sl] = (bx[p][r, sl] + by[p][r, sl]) * 0.5
            return ()

        lax.fori_loop(0, CHUNK, row_step, (), unroll=4)

        pend_out[p] = pltpu.async_copy(
            bx[p], out_hbm.at[pl.ds(base + c * CHUNK, CHUNK)], so[p]
        )
    for p in range(2):
        if pend_out[p] is not None:
            pend_out[p].wait()


@jax.jit
def _pe_lookup(xs, ys, pe):
    mesh = plsc.VectorSubcoreMesh(core_axis_name="c", subcore_axis_name="s")
    return pl.kernel(
        _sc_body,
        mesh=mesh,
        out_type=jax.ShapeDtypeStruct((N, EMBED_DIM), jnp.float32),
        scratch_types=[
            pltpu.VMEM((B_PER_W,), jnp.int32),
            pltpu.VMEM((B_PER_W,), jnp.int32),
            pltpu.VMEM((CHUNK, EMBED_DIM), jnp.float32),
            pltpu.VMEM((CHUNK, EMBED_DIM), jnp.float32),
            pltpu.VMEM((CHUNK, EMBED_DIM), jnp.float32),
            pltpu.VMEM((CHUNK, EMBED_DIM), jnp.float32),
            pltpu.SemaphoreType.DMA,
            pltpu.SemaphoreType.DMA,
            pltpu.SemaphoreType.DMA,
            pltpu.SemaphoreType.DMA,
            pltpu.SemaphoreType.DMA,
            pltpu.SemaphoreType.DMA,
        ],
    )(xs, ys, pe)


def kernel(coords, pe):
    xs = coords[:, 0].astype(jnp.int32)
    ys = coords[:, 1].astype(jnp.int32)
    return _pe_lookup(xs, ys, pe)


# speedup vs baseline: 2.6330x; 1.2319x over previous
import functools

import jax
import jax.numpy as jnp
from jax import lax
from jax.experimental import pallas as pl
from jax.experimental.pallas import tpu as pltpu
from jax.experimental.pallas import tpu_sc as plsc

EMBED_DIM = 128
MAX_LEN = 500
N = 16384

NUM_CORES = 2
NUM_SUBCORES = 16
NUM_WORKERS = NUM_CORES * NUM_SUBCORES
B_PER_W = N // NUM_WORKERS
CHUNK = 128
NCHUNKS = B_PER_W // CHUNK
LANES = 16
VECS_PER_ROW = EMBED_DIM // LANES


def _sc_body(
    xs_hbm, ys_hbm, pe_hbm, out_hbm, xs_v, ys_v,
    bx0, by0, bx1, by1, sx0, sy0, sx1, sy1, so0, so1,
):
    bx = (bx0, bx1)
    by = (by0, by1)
    sx = (sx0, sx1)
    sy = (sy0, sy1)
    so = (so0, so1)

    wid = lax.axis_index("s") * NUM_CORES + lax.axis_index("c")
    base = wid * B_PER_W
    pltpu.sync_copy(xs_hbm.at[pl.ds(base, B_PER_W)], xs_v)
    pltpu.sync_copy(ys_hbm.at[pl.ds(base, B_PER_W)], ys_v)

    def start_gathers(c):
        p = c % 2
        gx = pltpu.async_copy(
            pe_hbm.at[xs_v.at[pl.ds(c * CHUNK, CHUNK)]], bx[p], sx[p]
        )
        gy = pltpu.async_copy(
            pe_hbm.at[ys_v.at[pl.ds(c * CHUNK, CHUNK)]], by[p], sy[p]
        )
        return gx, gy

    pend_g = {0: start_gathers(0)}
    pend_out = [None, None]
    for c in range(NCHUNKS):
        p = c % 2
        if c + 1 < NCHUNKS:
            q = (c + 1) % 2
            if pend_out[q] is not None:
                pend_out[q].wait()
                pend_out[q] = None
            pend_g[c + 1] = start_gathers(c + 1)
        gx, gy = pend_g.pop(c)
        gx.wait()
        gy.wait()

        @plsc.parallel_loop(0, CHUNK, step=1, unroll=4)
        def row_step(r, p=p):
            for k in range(VECS_PER_ROW):
                sl = pl.ds(k * LANES, LANES)
                bx[p][r, sl] = (bx[p][r, sl] + by[p][r, sl]) * 0.5

        pend_out[p] = pltpu.async_copy(
            bx[p], out_hbm.at[pl.ds(base + c * CHUNK, CHUNK)], so[p]
        )
    for p in range(2):
        if pend_out[p] is not None:
            pend_out[p].wait()


@jax.jit
def _pe_lookup(xs, ys, pe):
    mesh = plsc.VectorSubcoreMesh(core_axis_name="c", subcore_axis_name="s")
    return pl.kernel(
        _sc_body,
        mesh=mesh,
        out_type=jax.ShapeDtypeStruct((N, EMBED_DIM), jnp.float32),
        scratch_types=[
            pltpu.VMEM((B_PER_W,), jnp.int32),
            pltpu.VMEM((B_PER_W,), jnp.int32),
            pltpu.VMEM((CHUNK, EMBED_DIM), jnp.float32),
            pltpu.VMEM((CHUNK, EMBED_DIM), jnp.float32),
            pltpu.VMEM((CHUNK, EMBED_DIM), jnp.float32),
            pltpu.VMEM((CHUNK, EMBED_DIM), jnp.float32),
            pltpu.SemaphoreType.DMA,
            pltpu.SemaphoreType.DMA,
            pltpu.SemaphoreType.DMA,
            pltpu.SemaphoreType.DMA,
            pltpu.SemaphoreType.DMA,
            pltpu.SemaphoreType.DMA,
        ],
    )(xs, ys, pe)


def kernel(coords, pe):
    xs = coords[:, 0].astype(jnp.int32)
    ys = coords[:, 1].astype(jnp.int32)
    return _pe_lookup(xs, ys, pe)


# speedup vs baseline: 3.7523x; 1.4251x over previous
import functools

import jax
import jax.numpy as jnp
from jax import lax
from jax.experimental import pallas as pl
from jax.experimental.pallas import tpu as pltpu
from jax.experimental.pallas import tpu_sc as plsc

EMBED_DIM = 128
MAX_LEN = 500
N = 16384

NUM_CORES = 2
NUM_SUBCORES = 16
NUM_WORKERS = NUM_CORES * NUM_SUBCORES
B_PER_W = N // NUM_WORKERS
CHUNK = 128
NCHUNKS = B_PER_W // CHUNK
LANES = 16
VECS_PER_ROW = EMBED_DIM // LANES
TAB_ROWS = 512


def _sc_body(
    xs_hbm, ys_hbm, pe_hbm, out_hbm, tab, xs_v, ys_v,
    bx0, by0, bx1, by1, sx0, sy0, sx1, sy1, so0, so1,
):
    bx = (bx0, bx1)
    by = (by0, by1)
    sx = (sx0, sx1)
    sy = (sy0, sy1)
    so = (so0, so1)

    sid = lax.axis_index("s")
    wid = sid * NUM_CORES + lax.axis_index("c")
    base = wid * B_PER_W

    rows_per_sub = TAB_ROWS // NUM_SUBCORES
    pltpu.sync_copy(
        pe_hbm.at[pl.ds(sid * rows_per_sub, rows_per_sub)],
        tab.at[pl.ds(sid * rows_per_sub, rows_per_sub)],
    )
    pltpu.sync_copy(xs_hbm.at[pl.ds(base, B_PER_W)], xs_v)
    pltpu.sync_copy(ys_hbm.at[pl.ds(base, B_PER_W)], ys_v)
    plsc.subcore_barrier()

    def start_gathers(c):
        p = c % 2
        gx = pltpu.async_copy(
            tab.at[xs_v.at[pl.ds(c * CHUNK, CHUNK)]], bx[p], sx[p]
        )
        gy = pltpu.async_copy(
            tab.at[ys_v.at[pl.ds(c * CHUNK, CHUNK)]], by[p], sy[p]
        )
        return gx, gy

    pend_g = {0: start_gathers(0)}
    pend_out = [None, None]
    for c in range(NCHUNKS):
        p = c % 2
        if c + 1 < NCHUNKS:
            q = (c + 1) % 2
            if pend_out[q] is not None:
                pend_out[q].wait()
                pend_out[q] = None
            pend_g[c + 1] = start_gathers(c + 1)
        gx, gy = pend_g.pop(c)
        gx.wait()
        gy.wait()

        @plsc.parallel_loop(0, CHUNK, step=1, unroll=4)
        def row_step(r, p=p):
            for k in range(VECS_PER_ROW):
                sl = pl.ds(k * LANES, LANES)
                bx[p][r, sl] = (bx[p][r, sl] + by[p][r, sl]) * 0.5

        pend_out[p] = pltpu.async_copy(
            bx[p], out_hbm.at[pl.ds(base + c * CHUNK, CHUNK)], so[p]
        )
    for p in range(2):
        if pend_out[p] is not None:
            pend_out[p].wait()


@jax.jit
def _pe_lookup(xs, ys, pe):
    mesh = plsc.VectorSubcoreMesh(core_axis_name="c", subcore_axis_name="s")
    return pl.kernel(
        _sc_body,
        mesh=mesh,
        out_type=jax.ShapeDtypeStruct((N, EMBED_DIM), jnp.float32),
        scratch_types=[
            pltpu.VMEM_SHARED((TAB_ROWS, EMBED_DIM), jnp.float32),
            pltpu.VMEM((B_PER_W,), jnp.int32),
            pltpu.VMEM((B_PER_W,), jnp.int32),
            pltpu.VMEM((CHUNK, EMBED_DIM), jnp.float32),
            pltpu.VMEM((CHUNK, EMBED_DIM), jnp.float32),
            pltpu.VMEM((CHUNK, EMBED_DIM), jnp.float32),
            pltpu.VMEM((CHUNK, EMBED_DIM), jnp.float32),
            pltpu.SemaphoreType.DMA,
            pltpu.SemaphoreType.DMA,
            pltpu.SemaphoreType.DMA,
            pltpu.SemaphoreType.DMA,
            pltpu.SemaphoreType.DMA,
            pltpu.SemaphoreType.DMA,
        ],
    )(xs, ys, pe)


def kernel(coords, pe):
    xs = coords[:, 0].astype(jnp.int32)
    ys = coords[:, 1].astype(jnp.int32)
    pe_pad = jnp.zeros((TAB_ROWS, EMBED_DIM), pe.dtype).at[:MAX_LEN].set(pe)
    return _pe_lookup(xs, ys, pe_pad)


# speedup vs baseline: 3.7617x; 1.0025x over previous
import functools

import jax
import jax.numpy as jnp
from jax import lax
from jax.experimental import pallas as pl
from jax.experimental.pallas import tpu as pltpu
from jax.experimental.pallas import tpu_sc as plsc

EMBED_DIM = 128
MAX_LEN = 500
N = 16384

NUM_CORES = 2
NUM_SUBCORES = 16
NUM_WORKERS = NUM_CORES * NUM_SUBCORES
B_PER_W = N // NUM_WORKERS
CHUNK = 128
NCHUNKS = B_PER_W // CHUNK
LANES = 16
VECS_PER_ROW = EMBED_DIM // LANES
TAB_ROWS = 512


def _sc_body(
    xs_hbm, ys_hbm, pe_hbm, out_hbm, tab, xs_v, ys_v,
    bx0, by0, bx1, by1, sx0, sy0, sx1, sy1, so0, so1,
):
    bx = (bx0, bx1)
    by = (by0, by1)
    sx = (sx0, sx1)
    sy = (sy0, sy1)
    so = (so0, so1)

    sid = lax.axis_index("s")
    wid = sid * NUM_CORES + lax.axis_index("c")
    base = wid * B_PER_W

    @pl.when(sid < NUM_SUBCORES - 1)
    def _copy_full_stripe():
        start = pl.multiple_of(sid * 32, 32)
        pltpu.sync_copy(pe_hbm.at[pl.ds(start, 32)], tab.at[pl.ds(start, 32)])

    @pl.when(sid == NUM_SUBCORES - 1)
    def _copy_tail_stripe():
        pltpu.sync_copy(pe_hbm.at[pl.ds(480, 20)], tab.at[pl.ds(480, 20)])
    pltpu.sync_copy(xs_hbm.at[pl.ds(base, B_PER_W)], xs_v)
    pltpu.sync_copy(ys_hbm.at[pl.ds(base, B_PER_W)], ys_v)
    plsc.subcore_barrier()

    def start_gathers(c):
        p = c % 2
        gx = pltpu.async_copy(
            tab.at[xs_v.at[pl.ds(c * CHUNK, CHUNK)]], bx[p], sx[p]
        )
        gy = pltpu.async_copy(
            tab.at[ys_v.at[pl.ds(c * CHUNK, CHUNK)]], by[p], sy[p]
        )
        return gx, gy

    pend_g = {0: start_gathers(0)}
    pend_out = [None, None]
    for c in range(NCHUNKS):
        p = c % 2
        if c + 1 < NCHUNKS:
            q = (c + 1) % 2
            if pend_out[q] is not None:
                pend_out[q].wait()
                pend_out[q] = None
            pend_g[c + 1] = start_gathers(c + 1)
        gx, gy = pend_g.pop(c)
        gx.wait()
        gy.wait()

        @plsc.parallel_loop(0, CHUNK, step=1, unroll=4)
        def row_step(r, p=p):
            for k in range(VECS_PER_ROW):
                sl = pl.ds(k * LANES, LANES)
                bx[p][r, sl] = (bx[p][r, sl] + by[p][r, sl]) * 0.5

        pend_out[p] = pltpu.async_copy(
            bx[p], out_hbm.at[pl.ds(base + c * CHUNK, CHUNK)], so[p]
        )
    for p in range(2):
        if pend_out[p] is not None:
            pend_out[p].wait()


@jax.jit
def _pe_lookup(xs, ys, pe):
    mesh = plsc.VectorSubcoreMesh(core_axis_name="c", subcore_axis_name="s")
    return pl.kernel(
        _sc_body,
        mesh=mesh,
        out_type=jax.ShapeDtypeStruct((N, EMBED_DIM), jnp.float32),
        scratch_types=[
            pltpu.VMEM_SHARED((MAX_LEN, EMBED_DIM), jnp.float32),
            pltpu.VMEM((B_PER_W,), jnp.int32),
            pltpu.VMEM((B_PER_W,), jnp.int32),
            pltpu.VMEM((CHUNK, EMBED_DIM), jnp.float32),
            pltpu.VMEM((CHUNK, EMBED_DIM), jnp.float32),
            pltpu.VMEM((CHUNK, EMBED_DIM), jnp.float32),
            pltpu.VMEM((CHUNK, EMBED_DIM), jnp.float32),
            pltpu.SemaphoreType.DMA,
            pltpu.SemaphoreType.DMA,
            pltpu.SemaphoreType.DMA,
            pltpu.SemaphoreType.DMA,
            pltpu.SemaphoreType.DMA,
            pltpu.SemaphoreType.DMA,
        ],
    )(xs, ys, pe)


def kernel(coords, pe):
    xs = coords[:, 0].astype(jnp.int32)
    ys = coords[:, 1].astype(jnp.int32)
    return _pe_lookup(xs, ys, pe)


# speedup vs baseline: 4.1423x; 1.1012x over previous
import functools

import jax
import jax.numpy as jnp
from jax import lax
from jax.experimental import pallas as pl
from jax.experimental.pallas import tpu as pltpu
from jax.experimental.pallas import tpu_sc as plsc

EMBED_DIM = 128
MAX_LEN = 500
N = 16384

NUM_CORES = 2
NUM_SUBCORES = 16
NUM_WORKERS = NUM_CORES * NUM_SUBCORES
B_PER_W = N // NUM_WORKERS
CHUNK = 128
NCHUNKS = B_PER_W // CHUNK
LANES = 16
VECS_PER_ROW = EMBED_DIM // LANES


def _sc_body(
    xs_hbm, ys_hbm, pe_hbm, out_hbm, tab, xs_v, ys_v, tmp,
    bx0, bx1, sg, so0, so1,
):
    bx = (bx0, bx1)
    so = (so0, so1)

    sid = lax.axis_index("s")
    wid = sid * NUM_CORES + lax.axis_index("c")
    base = wid * B_PER_W

    def scale_rows(nrows):
        def row_step(r, _):
            for k in range(VECS_PER_ROW):
                sl = pl.ds(k * LANES, LANES)
                tmp[r, sl] = tmp[r, sl] * 0.5
            return ()

        lax.fori_loop(0, nrows, row_step, ())

    @pl.when(sid < NUM_SUBCORES - 1)
    def _full_stripe():
        start = pl.multiple_of(sid * 32, 32)
        pltpu.sync_copy(pe_hbm.at[pl.ds(start, 32)], tmp)
        scale_rows(32)
        pltpu.sync_copy(tmp, tab.at[pl.ds(start, 32)])

    @pl.when(sid == NUM_SUBCORES - 1)
    def _tail_stripe():
        pltpu.sync_copy(pe_hbm.at[pl.ds(480, 20)], tmp.at[pl.ds(0, 20)])
        scale_rows(20)
        pltpu.sync_copy(tmp.at[pl.ds(0, 20)], tab.at[pl.ds(480, 20)])

    pltpu.sync_copy(xs_hbm.at[pl.ds(base, B_PER_W)], xs_v)
    pltpu.sync_copy(ys_hbm.at[pl.ds(base, B_PER_W)], ys_v)
    plsc.subcore_barrier()

    pend_out = [None, None]
    for c in range(NCHUNKS):
        p = c % 2
        if pend_out[p] is not None:
            pend_out[p].wait()
            pend_out[p] = None
        pltpu.async_copy(
            tab.at[xs_v.at[pl.ds(c * CHUNK, CHUNK)]], bx[p], sg
        ).wait()
        pltpu.async_copy(
            tab.at[ys_v.at[pl.ds(c * CHUNK, CHUNK)]], bx[p], sg, add=True
        ).wait()
        pend_out[p] = pltpu.async_copy(
            bx[p], out_hbm.at[pl.ds(base + c * CHUNK, CHUNK)], so[p]
        )
    for p in range(2):
        if pend_out[p] is not None:
            pend_out[p].wait()


@jax.jit
def _pe_lookup(xs, ys, pe):
    mesh = plsc.VectorSubcoreMesh(core_axis_name="c", subcore_axis_name="s")
    return pl.kernel(
        _sc_body,
        mesh=mesh,
        out_type=jax.ShapeDtypeStruct((N, EMBED_DIM), jnp.float32),
        scratch_types=[
            pltpu.VMEM_SHARED((MAX_LEN, EMBED_DIM), jnp.float32),
            pltpu.VMEM((B_PER_W,), jnp.int32),
            pltpu.VMEM((B_PER_W,), jnp.int32),
            pltpu.VMEM((32, EMBED_DIM), jnp.float32),
            pltpu.VMEM((CHUNK, EMBED_DIM), jnp.float32),
            pltpu.VMEM((CHUNK, EMBED_DIM), jnp.float32),
            pltpu.SemaphoreType.DMA,
            pltpu.SemaphoreType.DMA,
            pltpu.SemaphoreType.DMA,
        ],
    )(xs, ys, pe)


def kernel(coords, pe):
    xs = coords[:, 0].astype(jnp.int32)
    ys = coords[:, 1].astype(jnp.int32)
    return _pe_lookup(xs, ys, pe)
